# Initial kernel scaffold; baseline (speedup 1.0000x reference)
import jax
import jax.numpy as jnp
from jax.experimental import pallas as pl


def kernel(x, mask_matrix, norm1_w, norm1_b, qkv_w, qkv_b, rpb, proj_w, proj_b, norm2_w, norm2_b, gate_w, w1, b1, w2, b2):
    raise NotImplementedError("write your pallas kernel here")


# speedup vs baseline: 1.8913x; 1.8913x over previous
import functools

import jax
import jax.numpy as jnp
import numpy as np
from jax.experimental import pallas as pl
from jax.experimental.pallas import tpu as pltpu

DIM = 192
NH = 6
HD = DIM // NH
WS = (6, 8, 8)
N = WS[0] * WS[1] * WS[2]
NW = 36
E = 8
DH = 384
T = 6 * 48 * 48


def _rel_index_np(ws):
    wd, wh, ww = ws
    coords = np.stack(
        np.meshgrid(np.arange(wd), np.arange(wh), np.arange(ww), indexing="ij"))
    cf = coords.reshape(3, -1)
    rel = cf[:, :, None] - cf[:, None, :]
    rel = rel.transpose(1, 2, 0).astype(np.int64)
    rel[..., 0] += wd - 1
    rel[..., 1] += wh - 1
    rel[..., 2] += ww - 1
    rel[..., 0] *= (2 * wh - 1) * (2 * ww - 1)
    rel[..., 1] *= (2 * ww - 1)
    return rel.sum(-1)


_REL_FLAT = _rel_index_np(WS).reshape(-1)


def _ln(x, w, b):
    m = jnp.mean(x, axis=-1, keepdims=True)
    xc = x - m
    v = jnp.mean(xc * xc, axis=-1, keepdims=True)
    return xc * jax.lax.rsqrt(v + 1e-5) * w + b


def _attn_body(xw_ref, bias_ref, n1w_ref, n1b_ref, qkvwt_ref, qkvb_ref,
               projwt_ref, projb_ref, out_ref):
    x = xw_ref[0]
    xn = _ln(x, n1w_ref[0], n1b_ref[0])
    qkv = jnp.dot(xn, qkvwt_ref[...],
                  preferred_element_type=jnp.float32) + qkvb_ref[0]
    scale = HD ** -0.5
    outs = []
    for h in range(NH):
        q = qkv[:, h * HD:(h + 1) * HD] * scale
        k = qkv[:, DIM + h * HD:DIM + (h + 1) * HD]
        v = qkv[:, 2 * DIM + h * HD:2 * DIM + (h + 1) * HD]
        s = jnp.dot(q, k.T, preferred_element_type=jnp.float32) + bias_ref[h]
        s = s - jnp.max(s, axis=-1, keepdims=True)
        p = jnp.exp(s)
        p = p / jnp.sum(p, axis=-1, keepdims=True)
        outs.append(jnp.dot(p, v, preferred_element_type=jnp.float32))
    o = jnp.concatenate(outs, axis=-1)
    out_ref[0] = x + jnp.dot(o, projwt_ref[...],
                             preferred_element_type=jnp.float32) + projb_ref[0]


def _attention(xw, bias, norm1_w, norm1_b, qkv_w, qkv_b, proj_w, proj_b):
    qkv_wt = qkv_w.T
    proj_wt = proj_w.T
    return pl.pallas_call(
        _attn_body,
        grid=(NW,),
        in_specs=[
            pl.BlockSpec((1, N, DIM), lambda i: (i, 0, 0)),
            pl.BlockSpec((NH, N, N), lambda i: (0, 0, 0)),
            pl.BlockSpec((1, DIM), lambda i: (0, 0)),
            pl.BlockSpec((1, DIM), lambda i: (0, 0)),
            pl.BlockSpec((DIM, 3 * DIM), lambda i: (0, 0)),
            pl.BlockSpec((1, 3 * DIM), lambda i: (0, 0)),
            pl.BlockSpec((DIM, DIM), lambda i: (0, 0)),
            pl.BlockSpec((1, DIM), lambda i: (0, 0)),
        ],
        out_specs=pl.BlockSpec((1, N, DIM), lambda i: (i, 0, 0)),
        out_shape=jax.ShapeDtypeStruct((NW, N, DIM), jnp.float32),
    )(xw, bias, norm1_w.reshape(1, DIM), norm1_b.reshape(1, DIM), qkv_wt,
      qkv_b.reshape(1, 3 * DIM), proj_wt, proj_b.reshape(1, DIM))


def _gelu(x):
    return 0.5 * x * (1.0 + jax.lax.erf(x * (2.0 ** -0.5)))


def _top2(logits):
    M = logits.shape[0]
    m0 = jnp.full((M, 1), -jnp.inf, jnp.float32)
    e0 = jnp.zeros((M, 1), jnp.int32)
    for e in range(E):
        le = logits[:, e:e + 1]
        c = le > m0
        e0 = jnp.where(c, e, e0)
        m0 = jnp.where(c, le, m0)
    m1 = jnp.full((M, 1), -jnp.inf, jnp.float32)
    e1 = jnp.zeros((M, 1), jnp.int32)
    for e in range(E):
        le = logits[:, e:e + 1]
        c = (le > m1) & (e0 != e)
        e1 = jnp.where(c, e, e1)
        m1 = jnp.where(c, le, m1)
    g0 = 1.0 / (1.0 + jnp.exp(m1 - m0))
    g1 = 1.0 - g0
    return g0, g1, e0, e1


def _moe_dense_body(x1_ref, n2w_ref, n2b_ref, gwt_ref, w1t_ref, b1_ref,
                    w2t_ref, b2_ref, out_ref):
    x = x1_ref[...]
    t = _ln(x, n2w_ref[0], n2b_ref[0])
    logits = jnp.dot(t, gwt_ref[...], preferred_element_type=jnp.float32)
    g0, g1, e0, e1 = _top2(logits)
    acc = x
    for e in range(E):
        w_e = jnp.where(e0 == e, g0, 0.0) + jnp.where(e1 == e, g1, 0.0)
        h = _gelu(jnp.dot(t, w1t_ref[e], preferred_element_type=jnp.float32)
                  + b1_ref[0, e])
        y = jnp.dot(h, w2t_ref[e], preferred_element_type=jnp.float32) + b2_ref[0, e]
        acc = acc + w_e * y
    out_ref[...] = acc


def _moe_dense(x1, norm2_w, norm2_b, gate_w, w1, b1, w2, b2):
    TM = 1728
    w1t = jnp.transpose(w1, (0, 2, 1))
    w2t = jnp.transpose(w2, (0, 2, 1))
    return pl.pallas_call(
        _moe_dense_body,
        grid=(T // TM,),
        in_specs=[
            pl.BlockSpec((TM, DIM), lambda i: (i, 0)),
            pl.BlockSpec((1, DIM), lambda i: (0, 0)),
            pl.BlockSpec((1, DIM), lambda i: (0, 0)),
            pl.BlockSpec((DIM, E), lambda i: (0, 0)),
            pl.BlockSpec((E, DIM, DH), lambda i: (0, 0, 0)),
            pl.BlockSpec((1, E, DH), lambda i: (0, 0, 0)),
            pl.BlockSpec((E, DH, DIM), lambda i: (0, 0, 0)),
            pl.BlockSpec((1, E, DIM), lambda i: (0, 0, 0)),
        ],
        out_specs=pl.BlockSpec((TM, DIM), lambda i: (i, 0)),
        out_shape=jax.ShapeDtypeStruct((T, DIM), jnp.float32),
    )(x1, norm2_w.reshape(1, DIM), norm2_b.reshape(1, DIM), gate_w.T, w1t,
      b1.reshape(1, E, DH), w2t, b2.reshape(1, E, DIM))


def kernel(x, mask_matrix, norm1_w, norm1_b, qkv_w, qkv_b, rpb, proj_w,
           proj_b, norm2_w, norm2_b, gate_w, w1, b1, w2, b2):
    del mask_matrix
    B, D, H, W, C = x.shape
    wd, wh, ww = WS
    xw = x.reshape(B, D // wd, wd, H // wh, wh, W // ww, ww, C)
    xw = xw.transpose(0, 1, 3, 5, 2, 4, 6, 7).reshape(NW, N, C)
    bias = jnp.take(rpb, _REL_FLAT, axis=0).reshape(N, N, NH).transpose(2, 0, 1)
    x1w = _attention(xw, bias, norm1_w, norm1_b, qkv_w, qkv_b, proj_w, proj_b)
    x1 = x1w.reshape(B, D // wd, H // wh, W // ww, wd, wh, ww, C)
    x1 = x1.transpose(0, 1, 4, 2, 5, 3, 6, 7).reshape(B, D, H, W, C)
    out = _moe_dense(x1.reshape(T, C), norm2_w, norm2_b, gate_w, w1, b1, w2, b2)
    return out.reshape(B, D, H, W, C)
